# baseline (speedup 1.0000x reference)
import functools

import jax
import jax.numpy as jnp
from jax import lax
from jax.experimental import pallas as pl
from jax.experimental.pallas import tpu as pltpu
from jax.experimental.pallas import tpu_sc as plsc

B = 16384
F = 26
D = 32
GW = 128
NV = 1000001
NB = 7813
NB_MAIN = 7808
NGRP = NB * 32
NROW = NGRP * 4
NW = 32
SPW = B // NW
CH = 64
NCHUNK = SPW // CH
G = 128
NG = CH * F // G
ROWS = CH * F
UB = 2
UCOL = UB * 128
UGO = UB * 32
NUNIT = 122


def _tr_body(src_hbm, tail_hbm, out_hbm, tiles_v, outblk_v, tail_v,
             tail24_v, sem_in, sem_out):
    cid = lax.axis_index("c")
    sid = lax.axis_index("s")
    wid = cid * 16 + sid

    iota16 = lax.iota(jnp.int32, 16)

    def in_copies(u, q):
        c0 = pl.multiple_of(u * UCOL, 128)
        return [
            pltpu.make_async_copy(
                src_hbm.at[pl.ds(0, 32), pl.ds(c0, UCOL)],
                tiles_v.at[q], sem_in)
        ]

    def out_copy(u, q):
        g0 = pl.multiple_of(u * UGO, 32)
        return pltpu.make_async_copy(
            outblk_v.at[q], out_hbm.at[pl.ds(g0, UGO)], sem_out)

    def shuffle(src, dst, ncol):
        @plsc.parallel_loop(0, ncol, step=1, unroll=8)
        def col_body(c0):
            c16 = (iota16 + c0) & (ncol - 1)
            go16 = lax.shift_right_logical(c16, 2)
            pb = lax.shift_left(c16 & 3, 5)
            for h in range(2):
                vals = plsc.load_gather(src, [iota16 + h * 16, c16])
                plsc.store_scatter(
                    dst, [go16, pb + iota16 + h * 16], vals)

    for cp in in_copies(wid, 0):
        cp.start()

    def pair_body(i2, carry):
        for q in range(2):
            i = i2 * 2 + q
            u = i * 32 + wid
            if q == 0:
                for cp in in_copies(u + 32, 1):
                    cp.start()
            else:
                @pl.when(i2 < NUNIT // 2 - 1)
                def _():
                    for cp in in_copies(u + 32, 0):
                        cp.start()
            for cp in in_copies(u, q):
                cp.wait()

            @pl.when(i2 >= 1)
            def _():
                out_copy(u - 64, q).wait()

            shuffle(tiles_v.at[q], outblk_v.at[q], UCOL)
            out_copy(u, q).start()
        return carry

    lax.fori_loop(0, NUNIT // 2, pair_body, None)
    out_copy((NUNIT - 2) * 32 + wid, 0).wait()
    out_copy((NUNIT - 1) * 32 + wid, 1).wait()


    @pl.when(wid < 4)
    def _():
        b = NB_MAIN + wid
        c0 = pl.multiple_of(b * 128, 128)
        for t in range(4):
            pltpu.sync_copy(src_hbm.at[pl.ds(t * 8, 8), pl.ds(c0, 128)],
                            tail_v.at[pl.ds(t * 8, 8)])
        shuffle(tail_v, outblk_v.at[0], 128)
        g0 = pl.multiple_of(b * 32, 32)
        pltpu.sync_copy(outblk_v.at[0, pl.ds(0, 32)],
                        out_hbm.at[pl.ds(g0, 32)])

    @pl.when(wid == 4)
    def _():
        pltpu.sync_copy(tail_hbm, tail24_v)
        pltpu.sync_copy(tail24_v, out_hbm.at[pl.ds(7812 * 32, 24)])


_tr = functools.partial(
    pl.kernel,
    mesh=plsc.VectorSubcoreMesh(core_axis_name="c", subcore_axis_name="s"),
    out_type=jax.ShapeDtypeStruct((NGRP, GW), jnp.float32),
    scratch_types=[
        pltpu.VMEM((2, 32, UCOL), jnp.float32),
        pltpu.VMEM((2, UGO, GW), jnp.float32),
        pltpu.VMEM((32, 128), jnp.float32),
        pltpu.VMEM((24, 128), jnp.float32),
        pltpu.SemaphoreType.DMA,
        pltpu.SemaphoreType.DMA,
    ],
    compiler_params=pltpu.CompilerParams(
        needs_layout_passes=False, use_tc_tiling_on_sc=True
    ),
)(_tr_body)


def _fm_body(x_hbm, emb_hbm, lin_hbm, bias_hbm, out_hbm,
             idx_v, rows_v, lin_v, out_v, bias_v, sem, sem_i):
    cid = lax.axis_index("c")
    sid = lax.axis_index("s")
    wid = cid * 16 + sid

    pltpu.sync_copy(bias_hbm, bias_v.at[pl.ds(0, 1)])
    bias = bias_v[...][0]

    iota16 = lax.iota(jnp.int32, 16)
    lin_mask = iota16 < (F - 16)

    def idx_copy(c, q):
        gc = wid * NCHUNK + c
        return pltpu.make_async_copy(
            x_hbm.at[pl.ds(gc * ROWS, ROWS)],
            idx_v.at[q, pl.ds(0, ROWS)], sem_i)

    def gathers(c, q):
        def fire(j, carry2):
            pltpu.make_async_copy(
                emb_hbm.at[idx_v.at[q, pl.ds(j * G, G)]],
                rows_v.at[q, pl.ds(j * G, G)], sem
            ).start()
            pltpu.make_async_copy(
                lin_hbm.at[idx_v.at[q, pl.ds(j * G, G)]],
                lin_v.at[q, pl.ds(j * G, G)], sem
            ).start()
            return carry2

        lax.fori_loop(0, NG, fire, None)

    def drain(c, q):
        def drain_j(j, carry2):
            pltpu.make_async_copy(
                emb_hbm.at[idx_v.at[q, pl.ds(j * G, G)]],
                rows_v.at[q, pl.ds(j * G, G)], sem
            ).wait()
            pltpu.make_async_copy(
                lin_hbm.at[idx_v.at[q, pl.ds(j * G, G)]],
                lin_v.at[q, pl.ds(j * G, G)], sem
            ).wait()
            return carry2

        lax.fori_loop(0, NG, drain_j, None)

    def compute(c, q):
        def group_body(g, carry2):
            def sample_body(s16, vec):
                s = g * 16 + s16
                rb = s * F
                acc0 = jnp.zeros((16,), jnp.float32)
                acc1 = jnp.zeros((16,), jnp.float32)
                sq0 = jnp.zeros((16,), jnp.float32)
                sq1 = jnp.zeros((16,), jnp.float32)
                for f in range(F):
                    v0 = rows_v[q, rb + f, pl.ds(0, 16)]
                    v1 = rows_v[q, rb + f, pl.ds(16, 16)]
                    acc0 = acc0 + v0
                    acc1 = acc1 + v1
                    sq0 = sq0 + v0 * v0
                    sq1 = sq1 + v1 * v1
                l0 = lin_v[q, pl.ds(rb, 16)]
                l1 = lin_v[q, pl.ds(rb + 16, 16)]
                lin = jnp.sum(l0 + jnp.where(lin_mask, l1, 0.0))
                ps = jnp.sum(acc0 * acc0 + acc1 * acc1)
                sp = jnp.sum(sq0 + sq1)
                val = 0.5 * (ps - sp) + lin + bias
                return jnp.where(iota16 == s16, val, vec)

            vec = lax.fori_loop(0, 16, sample_body,
                                jnp.zeros((16,), jnp.float32))
            out_v[pl.ds(c * CH + g * 16, 16)] = vec
            return carry2

        lax.fori_loop(0, CH // 16, group_body, None)

    idx_copy(0, 0).start()
    idx_copy(0, 0).wait()
    gathers(0, 0)
    idx_copy(1, 1).start()

    def pair_body(i2, carry):
        for q in range(2):
            c = i2 * 2 + q
            nq = 1 - q

            @pl.when(c + 1 < NCHUNK)
            def _():
                idx_copy(c + 1, nq).wait()
                gathers(c + 1, nq)

            drain(c, q)

            @pl.when(c + 2 < NCHUNK)
            def _():
                idx_copy(c + 2, q).start()

            compute(c, q)
        return carry

    lax.fori_loop(0, NCHUNK // 2, pair_body, None)
    pltpu.sync_copy(out_v, out_hbm.at[pl.ds(wid * SPW, SPW)])


_fm = functools.partial(
    pl.kernel,
    mesh=plsc.VectorSubcoreMesh(core_axis_name="c", subcore_axis_name="s"),
    out_type=jax.ShapeDtypeStruct((B,), jnp.float32),
    scratch_types=[
        pltpu.VMEM((2, ROWS), jnp.int32),
        pltpu.VMEM((2, ROWS, D), jnp.float32),
        pltpu.VMEM((2, ROWS + 16), jnp.float32),
        pltpu.VMEM((SPW,), jnp.float32),
        pltpu.VMEM((16,), jnp.float32),
        pltpu.SemaphoreType.DMA,
        pltpu.SemaphoreType.DMA,
    ],
    compiler_params=pltpu.CompilerParams(
        needs_layout_passes=False, use_tc_tiling_on_sc=False
    ),
)(_fm_body)


def kernel(x, emb_w, lin_w, bias):
    x2 = x.astype(jnp.int32).reshape(B * F)
    tail = jnp.pad(emb_w[7812 * 128:], ((0, 31), (0, 0))).reshape(24, GW)
    emb_r = _tr(emb_w.T, tail).reshape(NROW, D)
    lin_flat = lin_w.reshape(-1)
    out = _fm(x2, emb_r, lin_flat, bias)
    return out.reshape(B, 1)


# speedup vs baseline: 1.1231x; 1.1231x over previous
import functools

import jax
import jax.numpy as jnp
from jax import lax
from jax.experimental import pallas as pl
from jax.experimental.pallas import tpu as pltpu
from jax.experimental.pallas import tpu_sc as plsc

B = 16384
F = 26
D = 32
GW = 128
NV = 1000001
NB = 7813
NB_MAIN = 7808
NGRP = NB * 32
NROW = NGRP * 4
NW = 32
SPW = B // NW
CH = 64
NCHUNK = SPW // CH
G = 128
NG = CH * F // G
ROWS = CH * F
UB = 4
UCOL = UB * 128
UGO = UB * 32
NUNIT = 61


def _tr_body(src_hbm, tail_hbm, out_hbm, tiles_v, outblk_v, tail_v,
             tail24_v, sem_in, sem_out):
    cid = lax.axis_index("c")
    sid = lax.axis_index("s")
    wid = cid * 16 + sid

    iota16 = lax.iota(jnp.int32, 16)

    def in_copies(u, q):
        c0 = pl.multiple_of(u * UCOL, 128)
        return [
            pltpu.make_async_copy(
                src_hbm.at[pl.ds(0, 32), pl.ds(c0, UCOL)],
                tiles_v.at[q], sem_in)
        ]

    def out_copy(u, q):
        g0 = pl.multiple_of(u * UGO, 32)
        return pltpu.make_async_copy(
            outblk_v.at[q], out_hbm.at[pl.ds(g0, UGO)], sem_out)

    def shuffle(src, dst, ncol):
        @plsc.parallel_loop(0, ncol, step=1, unroll=8)
        def col_body(c0):
            c16 = (iota16 + c0) & (ncol - 1)
            go16 = lax.shift_right_logical(c16, 2)
            pb = lax.shift_left(c16 & 3, 5)
            for h in range(2):
                vals = plsc.load_gather(src, [iota16 + h * 16, c16])
                plsc.store_scatter(
                    dst, [go16, pb + iota16 + h * 16], vals)

    for cp in in_copies(wid, 0):
        cp.start()

    def pair_body(i2, carry):
        for q in range(2):
            i = i2 * 2 + q
            u = i * 32 + wid
            for cp in in_copies(u + 32, 1 - q):
                cp.start()
            for cp in in_copies(u, q):
                cp.wait()

            @pl.when(i2 >= 1)
            def _():
                out_copy(u - 64, q).wait()

            shuffle(tiles_v.at[q], outblk_v.at[q], UCOL)
            out_copy(u, q).start()
        return carry

    lax.fori_loop(0, NUNIT // 2, pair_body, None)

    u_last = (NUNIT - 1) * 32 + wid
    for cp in in_copies(u_last, 0):
        cp.wait()
    out_copy((NUNIT - 3) * 32 + wid, 0).wait()
    shuffle(tiles_v.at[0], outblk_v.at[0], UCOL)
    out_copy(u_last, 0).start()
    out_copy((NUNIT - 2) * 32 + wid, 1).wait()
    out_copy(u_last, 0).wait()


    @pl.when(wid < 4)
    def _():
        b = NB_MAIN + wid
        c0 = pl.multiple_of(b * 128, 128)
        for t in range(4):
            pltpu.sync_copy(src_hbm.at[pl.ds(t * 8, 8), pl.ds(c0, 128)],
                            tail_v.at[pl.ds(t * 8, 8)])
        shuffle(tail_v, outblk_v.at[0], 128)
        g0 = pl.multiple_of(b * 32, 32)
        pltpu.sync_copy(outblk_v.at[0, pl.ds(0, 32)],
                        out_hbm.at[pl.ds(g0, 32)])

    @pl.when(wid == 4)
    def _():
        pltpu.sync_copy(tail_hbm, tail24_v)
        pltpu.sync_copy(tail24_v, out_hbm.at[pl.ds(7812 * 32, 24)])


_tr = functools.partial(
    pl.kernel,
    mesh=plsc.VectorSubcoreMesh(core_axis_name="c", subcore_axis_name="s"),
    out_type=jax.ShapeDtypeStruct((NGRP, GW), jnp.float32),
    scratch_types=[
        pltpu.VMEM((2, 32, UCOL), jnp.float32),
        pltpu.VMEM((2, UGO, GW), jnp.float32),
        pltpu.VMEM((32, 128), jnp.float32),
        pltpu.VMEM((24, 128), jnp.float32),
        pltpu.SemaphoreType.DMA,
        pltpu.SemaphoreType.DMA,
    ],
    compiler_params=pltpu.CompilerParams(
        needs_layout_passes=False, use_tc_tiling_on_sc=True
    ),
)(_tr_body)


def _fm_body(x_hbm, emb_hbm, lin_hbm, bias_hbm, out_hbm,
             idx_v, rows_v, lin_v, out_v, bias_v, sem, sem_i):
    cid = lax.axis_index("c")
    sid = lax.axis_index("s")
    wid = cid * 16 + sid

    pltpu.sync_copy(bias_hbm, bias_v.at[pl.ds(0, 1)])
    bias = bias_v[...][0]

    iota16 = lax.iota(jnp.int32, 16)
    lin_mask = iota16 < (F - 16)

    def idx_copy(c, q):
        gc = wid * NCHUNK + c
        return pltpu.make_async_copy(
            x_hbm.at[pl.ds(gc * ROWS, ROWS)],
            idx_v.at[q, pl.ds(0, ROWS)], sem_i)

    def gathers(c, q):
        def fire(j, carry2):
            pltpu.make_async_copy(
                emb_hbm.at[idx_v.at[q, pl.ds(j * G, G)]],
                rows_v.at[q, pl.ds(j * G, G)], sem
            ).start()
            pltpu.make_async_copy(
                lin_hbm.at[idx_v.at[q, pl.ds(j * G, G)]],
                lin_v.at[q, pl.ds(j * G, G)], sem
            ).start()
            return carry2

        lax.fori_loop(0, NG, fire, None)

    def drain(c, q):
        def drain_j(j, carry2):
            pltpu.make_async_copy(
                emb_hbm.at[idx_v.at[q, pl.ds(j * G, G)]],
                rows_v.at[q, pl.ds(j * G, G)], sem
            ).wait()
            pltpu.make_async_copy(
                lin_hbm.at[idx_v.at[q, pl.ds(j * G, G)]],
                lin_v.at[q, pl.ds(j * G, G)], sem
            ).wait()
            return carry2

        lax.fori_loop(0, NG, drain_j, None)

    def compute(c, q):
        def group_body(g, carry2):
            def sample_body(s16, vec):
                s = g * 16 + s16
                rb = s * F
                acc0 = jnp.zeros((16,), jnp.float32)
                acc1 = jnp.zeros((16,), jnp.float32)
                sq0 = jnp.zeros((16,), jnp.float32)
                sq1 = jnp.zeros((16,), jnp.float32)
                for f in range(F):
                    v0 = rows_v[q, rb + f, pl.ds(0, 16)]
                    v1 = rows_v[q, rb + f, pl.ds(16, 16)]
                    acc0 = acc0 + v0
                    acc1 = acc1 + v1
                    sq0 = sq0 + v0 * v0
                    sq1 = sq1 + v1 * v1
                l0 = lin_v[q, pl.ds(rb, 16)]
                l1 = lin_v[q, pl.ds(rb + 16, 16)]
                lin = jnp.sum(l0 + jnp.where(lin_mask, l1, 0.0))
                ps = jnp.sum(acc0 * acc0 + acc1 * acc1)
                sp = jnp.sum(sq0 + sq1)
                val = 0.5 * (ps - sp) + lin + bias
                return jnp.where(iota16 == s16, val, vec)

            vec = lax.fori_loop(0, 16, sample_body,
                                jnp.zeros((16,), jnp.float32))
            out_v[pl.ds(c * CH + g * 16, 16)] = vec
            return carry2

        lax.fori_loop(0, CH // 16, group_body, None)

    idx_copy(0, 0).start()
    idx_copy(0, 0).wait()
    gathers(0, 0)
    idx_copy(1, 1).start()

    def pair_body(i2, carry):
        for q in range(2):
            c = i2 * 2 + q
            nq = 1 - q

            @pl.when(c + 1 < NCHUNK)
            def _():
                idx_copy(c + 1, nq).wait()
                gathers(c + 1, nq)

            drain(c, q)

            @pl.when(c + 2 < NCHUNK)
            def _():
                idx_copy(c + 2, q).start()

            compute(c, q)
        return carry

    lax.fori_loop(0, NCHUNK // 2, pair_body, None)
    pltpu.sync_copy(out_v, out_hbm.at[pl.ds(wid * SPW, SPW)])


_fm = functools.partial(
    pl.kernel,
    mesh=plsc.VectorSubcoreMesh(core_axis_name="c", subcore_axis_name="s"),
    out_type=jax.ShapeDtypeStruct((B,), jnp.float32),
    scratch_types=[
        pltpu.VMEM((2, ROWS), jnp.int32),
        pltpu.VMEM((2, ROWS, D), jnp.float32),
        pltpu.VMEM((2, ROWS + 16), jnp.float32),
        pltpu.VMEM((SPW,), jnp.float32),
        pltpu.VMEM((16,), jnp.float32),
        pltpu.SemaphoreType.DMA,
        pltpu.SemaphoreType.DMA,
    ],
    compiler_params=pltpu.CompilerParams(
        needs_layout_passes=False, use_tc_tiling_on_sc=False
    ),
)(_fm_body)


def kernel(x, emb_w, lin_w, bias):
    x2 = x.astype(jnp.int32).reshape(B * F)
    tail = jnp.pad(emb_w[7812 * 128:], ((0, 31), (0, 0))).reshape(24, GW)
    emb_r = _tr(emb_w.T, tail).reshape(NROW, D)
    lin_flat = lin_w.reshape(-1)
    out = _fm(x2, emb_r, lin_flat, bias)
    return out.reshape(B, 1)
